# baseline (speedup 1.0000x reference)
import functools

import jax
import jax.numpy as jnp
from jax import lax
from jax.experimental import pallas as pl
from jax.experimental.pallas import tpu as pltpu
from jax.experimental.pallas import tpu_sc as plsc

B = 16384
N_NUM = 13
NUM_OUT = 64
N_CAT = 26
VOCAB = 1000
EMB = 32
OUT_D = NUM_OUT + N_CAT * EMB

NC = 2
NS = 16
NW = NC * NS
NSPLIT = 2
BSPLIT = B // NSPLIT
ROWS_PER_W = BSPLIT // NW
CHUNK = 128
N_CHUNKS = ROWS_PER_W // CHUNK
LANES = 16


def _mm_body(x_ref, w_ref, b_ref, o_ref):
    o_ref[...] = (
        jnp.dot(x_ref[...], w_ref[...], preferred_element_type=jnp.float32)
        + b_ref[...]
    )


def _num_matmul(x, W, b2):
    MB = 2048
    return pl.pallas_call(
        _mm_body,
        grid=(B // MB,),
        in_specs=[
            pl.BlockSpec((MB, N_NUM), lambda i: (i, 0)),
            pl.BlockSpec((N_NUM, NUM_OUT), lambda i: (0, 0)),
            pl.BlockSpec((1, NUM_OUT), lambda i: (0, 0)),
        ],
        out_specs=pl.BlockSpec((MB, NUM_OUT), lambda i: (i, 0)),
        out_shape=jax.ShapeDtypeStruct((B, NUM_OUT), jnp.float32),
    )(x, W, b2)


def _sc_body(split, num_emb_hbm, catT_hbm, tables_hbm, out_hbm,
             idx_v, dest_v, num_v, gsem, osem, ssem):
    cid = lax.axis_index("c")
    sid = lax.axis_index("s")
    wid = sid * NC + cid
    row0 = split * BSPLIT + wid * ROWS_PER_W

    def chunk_body(ci, carry):
        base = pl.multiple_of(row0 + ci * CHUNK, CHUNK)
        obase = pl.multiple_of(base - split * BSPLIT, CHUNK)
        pltpu.sync_copy(catT_hbm.at[:, pl.ds(base, CHUNK)], idx_v)
        for f in range(N_CAT):
            off = f * VOCAB
            for j in range(CHUNK // LANES):
                sl = pl.ds(j * LANES, LANES)
                idx_v[f, sl] = idx_v[f, sl] + off
        gathers = [
            pltpu.async_copy(tables_hbm.at[idx_v.at[f]], dest_v.at[f], gsem)
            for f in range(N_CAT)
        ]
        pltpu.async_copy(num_emb_hbm.at[pl.ds(base, CHUNK)], num_v, ssem).wait()
        out_num = pltpu.async_copy(
            num_v, out_hbm.at[pl.ds(obase, CHUNK), pl.ds(0, NUM_OUT)], ssem
        )
        outs = []
        for f in range(N_CAT):
            gathers[f].wait()
            outs.append(
                pltpu.async_copy(
                    dest_v.at[f],
                    out_hbm.at[
                        pl.ds(obase, CHUNK), pl.ds(NUM_OUT + f * EMB, EMB)
                    ],
                    osem,
                )
            )
        out_num.wait()
        for o in outs:
            o.wait()
        return carry

    lax.fori_loop(0, N_CHUNKS, chunk_body, 0)


def _make_sc_kernel(split):
    return pl.kernel(
        functools.partial(_sc_body, split),
        mesh=plsc.VectorSubcoreMesh(core_axis_name="c", subcore_axis_name="s"),
        compiler_params=pltpu.CompilerParams(
            use_tc_tiling_on_sc=False, needs_layout_passes=False
        ),
        out_type=jax.ShapeDtypeStruct((BSPLIT, OUT_D), jnp.float32),
        scratch_types=[
            pltpu.VMEM((N_CAT, CHUNK), jnp.int32),
            pltpu.VMEM((N_CAT, CHUNK, EMB), jnp.float32),
            pltpu.VMEM((CHUNK, NUM_OUT), jnp.float32),
            pltpu.SemaphoreType.DMA,
            pltpu.SemaphoreType.DMA,
            pltpu.SemaphoreType.DMA,
        ],
    )


_sc_kernels = [_make_sc_kernel(s) for s in range(NSPLIT)]


@jax.jit
def kernel(num_tensor, cat_tensor, W, b, tables):
    num_emb = _num_matmul(num_tensor, W, b.reshape(1, NUM_OUT))
    catT = cat_tensor.T
    tables_flat = tables.reshape(N_CAT * VOCAB, EMB)
    pieces = [
        k(num_emb, catT, tables_flat) for k in _sc_kernels
    ]
    out = jnp.empty((B, OUT_D), jnp.float32)
    for i, p in enumerate(pieces):
        out = lax.dynamic_update_slice(out, p, (i * BSPLIT, 0))
    return out


# speedup vs baseline: 1.2542x; 1.2542x over previous
import jax
import jax.numpy as jnp
from jax import lax
from jax.experimental import pallas as pl
from jax.experimental.pallas import tpu as pltpu
from jax.experimental.pallas import tpu_sc as plsc

B = 16384
N_NUM = 13
NUM_OUT = 64
N_CAT = 26
VOCAB = 1000
EMB = 32
OUT_D = NUM_OUT + N_CAT * EMB

NC = 2
NS = 16
NW = NC * NS
ROWS_PER_W = B // NW
CHUNK = 64
N_CHUNKS = ROWS_PER_W // CHUNK
LANES = 16


def _mm_body(x_ref, w_ref, b_ref, o_ref):
    o_ref[...] = (
        jnp.dot(x_ref[...], w_ref[...], preferred_element_type=jnp.float32)
        + b_ref[...]
    )


def _num_matmul(x, W, b2):
    MB = 2048
    return pl.pallas_call(
        _mm_body,
        grid=(B // MB,),
        in_specs=[
            pl.BlockSpec((MB, N_NUM), lambda i: (i, 0)),
            pl.BlockSpec((N_NUM, NUM_OUT), lambda i: (0, 0)),
            pl.BlockSpec((1, NUM_OUT), lambda i: (0, 0)),
        ],
        out_specs=pl.BlockSpec((MB, NUM_OUT), lambda i: (i, 0)),
        out_shape=jax.ShapeDtypeStruct((B, NUM_OUT), jnp.float32),
    )(x, W, b2)


def _sc_body(num_emb_hbm, catT_hbm, tables_hbm, out_hbm,
             idx0_v, idx1_v, dest0_v, dest1_v, num0_v, num1_v,
             gsem0, gsem1, osem0, osem1, nsem0, nsem1):
    cid = lax.axis_index("c")
    sid = lax.axis_index("s")
    wid = sid * NC + cid
    row0 = wid * ROWS_PER_W

    bufs = [
        (idx0_v, dest0_v, num0_v, gsem0, osem0, nsem0),
        (idx1_v, dest1_v, num1_v, gsem1, osem1, nsem1),
    ]

    def chunk_base(ci):
        return pl.multiple_of(row0 + ci * CHUNK, CHUNK)

    def fire(p, ci, first):
        idx_v, dest_v, num_v, gsem, osem, nsem = bufs[p]
        base = chunk_base(ci)
        pltpu.sync_copy(catT_hbm.at[:, pl.ds(base, CHUNK)], idx_v)
        for f in range(N_CAT):
            off = f * VOCAB
            for j in range(CHUNK // LANES):
                sl = pl.ds(j * LANES, LANES)
                idx_v[f, sl] = idx_v[f, sl] + off
        if not first:
            base_prev = chunk_base(ci - 2)
            pltpu.make_async_copy(
                num_v, out_hbm.at[pl.ds(base_prev, CHUNK), pl.ds(0, NUM_OUT)],
                osem,
            ).wait()
            for f in range(N_CAT):
                col = NUM_OUT + f * EMB
                pltpu.make_async_copy(
                    dest_v.at[f],
                    out_hbm.at[pl.ds(base_prev, CHUNK), pl.ds(col, EMB)],
                    osem,
                ).wait()
        for f in range(N_CAT):
            pltpu.async_copy(tables_hbm.at[idx_v.at[f]], dest_v.at[f], gsem)
        pltpu.async_copy(num_emb_hbm.at[pl.ds(base, CHUNK)], num_v, nsem)

    def drain_and_emit(p, ci):
        idx_v, dest_v, num_v, gsem, osem, nsem = bufs[p]
        base = chunk_base(ci)
        pltpu.make_async_copy(
            num_emb_hbm.at[pl.ds(base, CHUNK)], num_v, nsem
        ).wait()
        pltpu.async_copy(
            num_v, out_hbm.at[pl.ds(base, CHUNK), pl.ds(0, NUM_OUT)], osem
        )
        for f in range(N_CAT):
            pltpu.make_async_copy(
                tables_hbm.at[idx_v.at[f]], dest_v.at[f], gsem
            ).wait()
            col = NUM_OUT + f * EMB
            pltpu.async_copy(
                dest_v.at[f],
                out_hbm.at[pl.ds(base, CHUNK), pl.ds(col, EMB)],
                osem,
            )

    fire(0, 0, True)

    def pair_body(t, carry):
        @pl.when(t == 0)
        def _():
            fire(1, 1, True)

        @pl.when(t > 0)
        def _():
            fire(1, 2 * t + 1, False)

        drain_and_emit(0, 2 * t)

        @pl.when(t < N_CHUNKS // 2 - 1)
        def _():
            fire(0, 2 * t + 2, False)

        drain_and_emit(1, 2 * t + 1)
        return carry

    lax.fori_loop(0, N_CHUNKS // 2, pair_body, 0)

    for p, ci in ((0, N_CHUNKS - 2), (1, N_CHUNKS - 1)):
        idx_v, dest_v, num_v, gsem, osem, nsem = bufs[p]
        base = chunk_base(ci)
        pltpu.make_async_copy(
            num_v, out_hbm.at[pl.ds(base, CHUNK), pl.ds(0, NUM_OUT)], osem
        ).wait()
        for f in range(N_CAT):
            col = NUM_OUT + f * EMB
            pltpu.make_async_copy(
                dest_v.at[f],
                out_hbm.at[pl.ds(base, CHUNK), pl.ds(col, EMB)],
                osem,
            ).wait()


_sc_kernel = pl.kernel(
    _sc_body,
    mesh=plsc.VectorSubcoreMesh(core_axis_name="c", subcore_axis_name="s"),
    compiler_params=pltpu.CompilerParams(
        use_tc_tiling_on_sc=False, needs_layout_passes=False
    ),
    out_type=jax.ShapeDtypeStruct((B, OUT_D), jnp.float32),
    scratch_types=[
        pltpu.VMEM((N_CAT, CHUNK), jnp.int32),
        pltpu.VMEM((N_CAT, CHUNK), jnp.int32),
        pltpu.VMEM((N_CAT, CHUNK, EMB), jnp.float32),
        pltpu.VMEM((N_CAT, CHUNK, EMB), jnp.float32),
        pltpu.VMEM((CHUNK, NUM_OUT), jnp.float32),
        pltpu.VMEM((CHUNK, NUM_OUT), jnp.float32),
        pltpu.SemaphoreType.DMA,
        pltpu.SemaphoreType.DMA,
        pltpu.SemaphoreType.DMA,
        pltpu.SemaphoreType.DMA,
        pltpu.SemaphoreType.DMA,
        pltpu.SemaphoreType.DMA,
    ],
)


@jax.jit
def kernel(num_tensor, cat_tensor, W, b, tables):
    num_emb = _num_matmul(num_tensor, W, b.reshape(1, NUM_OUT))
    catT = cat_tensor.T
    tables_flat = tables.reshape(N_CAT * VOCAB, EMB)
    return _sc_kernel(num_emb, catT, tables_flat)


# speedup vs baseline: 1.3541x; 1.0797x over previous
import jax
import jax.numpy as jnp
from jax import lax
from jax.experimental import pallas as pl
from jax.experimental.pallas import tpu as pltpu
from jax.experimental.pallas import tpu_sc as plsc

B = 16384
N_NUM = 13
NUM_OUT = 64
N_CAT = 26
VOCAB = 1000
EMB = 32
OUT_D = NUM_OUT + N_CAT * EMB

NC = 2
NS = 16
NW = NC * NS
ROWS_PER_W = B // NW
CHUNK = 64
N_CHUNKS = ROWS_PER_W // CHUNK
LANES = 16


def _mm_body(x_ref, w_ref, b_ref, prev_ref, o_ref):
    mm = (
        jnp.dot(x_ref[...], w_ref[...], preferred_element_type=jnp.float32)
        + b_ref[...]
    )
    o_ref[...] = jnp.concatenate([mm, prev_ref[:, NUM_OUT:]], axis=1)


def _num_matmul_into(x, W, b2, out):
    MB = 2048
    return pl.pallas_call(
        _mm_body,
        grid=(B // MB,),
        in_specs=[
            pl.BlockSpec((MB, N_NUM), lambda i: (i, 0)),
            pl.BlockSpec((N_NUM, NUM_OUT), lambda i: (0, 0)),
            pl.BlockSpec((1, NUM_OUT), lambda i: (0, 0)),
            pl.BlockSpec((MB, 128), lambda i: (i, 0)),
        ],
        out_specs=pl.BlockSpec((MB, 128), lambda i: (i, 0)),
        out_shape=jax.ShapeDtypeStruct((B, OUT_D), jnp.float32),
        input_output_aliases={3: 0},
    )(x, W, b2, out)


def _sc_body(catT_hbm, tables_hbm, out_hbm,
             idx0_v, idx1_v, dest0_v, dest1_v,
             gsem0, gsem1, osem0, osem1):
    cid = lax.axis_index("c")
    sid = lax.axis_index("s")
    wid = sid * NC + cid
    row0 = wid * ROWS_PER_W

    bufs = [
        (idx0_v, dest0_v, gsem0, osem0),
        (idx1_v, dest1_v, gsem1, osem1),
    ]

    def chunk_base(ci):
        return pl.multiple_of(row0 + ci * CHUNK, CHUNK)

    def fire(p, ci, first):
        idx_v, dest_v, gsem, osem = bufs[p]
        base = chunk_base(ci)
        pltpu.sync_copy(catT_hbm.at[:, pl.ds(base, CHUNK)], idx_v)
        for f in range(N_CAT):
            off = f * VOCAB
            for j in range(CHUNK // LANES):
                sl = pl.ds(j * LANES, LANES)
                idx_v[f, sl] = idx_v[f, sl] + off
        if not first:
            base_prev = chunk_base(ci - 2)
            for f in range(N_CAT):
                col = NUM_OUT + f * EMB
                pltpu.make_async_copy(
                    dest_v.at[f],
                    out_hbm.at[pl.ds(base_prev, CHUNK), pl.ds(col, EMB)],
                    osem,
                ).wait()
        for f in range(N_CAT):
            pltpu.async_copy(tables_hbm.at[idx_v.at[f]], dest_v.at[f], gsem)

    def drain_and_emit(p, ci):
        idx_v, dest_v, gsem, osem = bufs[p]
        base = chunk_base(ci)
        for f in range(N_CAT):
            pltpu.make_async_copy(
                tables_hbm.at[idx_v.at[f]], dest_v.at[f], gsem
            ).wait()
            col = NUM_OUT + f * EMB
            pltpu.async_copy(
                dest_v.at[f],
                out_hbm.at[pl.ds(base, CHUNK), pl.ds(col, EMB)],
                osem,
            )

    fire(0, 0, True)

    def pair_body(t, carry):
        @pl.when(t == 0)
        def _():
            fire(1, 1, True)

        @pl.when(t > 0)
        def _():
            fire(1, 2 * t + 1, False)

        drain_and_emit(0, 2 * t)

        @pl.when(t < N_CHUNKS // 2 - 1)
        def _():
            fire(0, 2 * t + 2, False)

        drain_and_emit(1, 2 * t + 1)
        return carry

    lax.fori_loop(0, N_CHUNKS // 2, pair_body, 0)

    for p, ci in ((0, N_CHUNKS - 2), (1, N_CHUNKS - 1)):
        idx_v, dest_v, gsem, osem = bufs[p]
        base = chunk_base(ci)
        for f in range(N_CAT):
            col = NUM_OUT + f * EMB
            pltpu.make_async_copy(
                dest_v.at[f],
                out_hbm.at[pl.ds(base, CHUNK), pl.ds(col, EMB)],
                osem,
            ).wait()


_sc_kernel = pl.kernel(
    _sc_body,
    mesh=plsc.VectorSubcoreMesh(core_axis_name="c", subcore_axis_name="s"),
    compiler_params=pltpu.CompilerParams(
        use_tc_tiling_on_sc=False, needs_layout_passes=False
    ),
    out_type=jax.ShapeDtypeStruct((B, OUT_D), jnp.float32),
    scratch_types=[
        pltpu.VMEM((N_CAT, CHUNK), jnp.int32),
        pltpu.VMEM((N_CAT, CHUNK), jnp.int32),
        pltpu.VMEM((N_CAT, CHUNK, EMB), jnp.float32),
        pltpu.VMEM((N_CAT, CHUNK, EMB), jnp.float32),
        pltpu.SemaphoreType.DMA,
        pltpu.SemaphoreType.DMA,
        pltpu.SemaphoreType.DMA,
        pltpu.SemaphoreType.DMA,
    ],
)


@jax.jit
def kernel(num_tensor, cat_tensor, W, b, tables):
    catT = cat_tensor.T
    tables_flat = tables.reshape(N_CAT * VOCAB, EMB)
    cat_out = _sc_kernel(catT, tables_flat)
    return _num_matmul_into(num_tensor, W, b.reshape(1, NUM_OUT), cat_out)
